# baseline (speedup 1.0000x reference)
import functools

import jax
import jax.numpy as jnp
from jax import lax
from jax.experimental import pallas as pl
from jax.experimental.pallas import tpu as pltpu
from jax.experimental.pallas import tpu_sc as plsc

EPS = 1e-12

NC = 2
NS = 16
LN = 16
CH = 128


def _rsqrt_newton(v):
    i = plsc.bitcast(v, jnp.int32)
    y = plsc.bitcast(jnp.int32(0x5F3759DF) - lax.shift_right_arithmetic(i, 1),
                     jnp.float32)
    for _ in range(3):
        y = y * (1.5 - 0.5 * v * y * y)
    return y


def _sc_body(per_w, n_chunks, x_hbm, tags_hbm, epack_hbm, gamma_hbm, beta_hbm,
             out_hbm, xbuf, obuf, tagbuf, ebuf, gbuf, bbuf, scratch):
    wid = lax.axis_index("s") * NC + lax.axis_index("c")
    base = wid * per_w
    pltpu.sync_copy(epack_hbm, ebuf)
    pltpu.sync_copy(gamma_hbm, gbuf)
    pltpu.sync_copy(beta_hbm, bbuf)
    lanes = lax.iota(jnp.int32, LN)

    def chunk_body(ci, _):
        tok0 = base + ci * CH
        pltpu.sync_copy(x_hbm.at[pl.ds(tok0, CH), :], xbuf)
        pltpu.sync_copy(tags_hbm.at[pl.ds(tok0, CH)], tagbuf)

        def group_body(g, _):
            rows = g * LN + lanes
            tagv = plsc.load_gather(tagbuf, [rows])

            def d1(d, carry):
                acc, accsq = carry
                dfull = jnp.broadcast_to(d, (LN,))
                xv = plsc.load_gather(xbuf, [rows, dfull])
                ev = plsc.load_gather(ebuf, [tagv + 2 * d])
                hv = xv + ev
                plsc.store_scatter(scratch, [dfull, lanes], hv)
                return acc + hv, accsq + hv * hv

            zero = jnp.zeros((LN,), jnp.float32)
            acc, accsq = lax.fori_loop(0, 128, d1, (zero, zero))
            mean = acc * (1.0 / 128.0)
            var = accsq * (1.0 / 128.0) - mean * mean
            rstd = _rsqrt_newton(var + EPS)

            def d2(d, _):
                dfull = jnp.broadcast_to(d, (LN,))
                hv = plsc.load_gather(scratch, [dfull, lanes])
                gd = plsc.load_gather(gbuf, [dfull])
                bd = plsc.load_gather(bbuf, [dfull])
                ov = (hv - mean) * rstd * gd + bd
                plsc.store_scatter(obuf, [rows, dfull], ov)
                return 0

            lax.fori_loop(0, 128, d2, 0)
            return 0

        lax.fori_loop(0, CH // LN, group_body, 0)
        pltpu.sync_copy(obuf, out_hbm.at[pl.ds(tok0, CH), :])
        return 0

    lax.fori_loop(0, n_chunks, chunk_body, 0)


def _sc_call(x2, tagsi, epack, ln_gamma, ln_beta):
    N, D = x2.shape
    NW = NC * NS
    per_w = N // NW
    n_chunks = per_w // CH
    body = functools.partial(_sc_body, per_w, n_chunks)
    f = pl.kernel(
        body,
        mesh=plsc.VectorSubcoreMesh(core_axis_name="c", subcore_axis_name="s"),
        compiler_params=pltpu.CompilerParams(needs_layout_passes=False),
        out_type=jax.ShapeDtypeStruct((N, D), jnp.float32),
        scratch_types=[
            pltpu.VMEM((CH, D), jnp.float32),
            pltpu.VMEM((CH, D), jnp.float32),
            pltpu.VMEM((CH,), jnp.int32),
            pltpu.VMEM((2 * D,), jnp.float32),
            pltpu.VMEM((D,), jnp.float32),
            pltpu.VMEM((D,), jnp.float32),
            pltpu.VMEM((D, LN), jnp.float32),
        ],
    )
    return f(x2, tagsi, epack, ln_gamma, ln_beta)


def kernel(x, emotion_tags, emb_table, ln_gamma, ln_beta):
    B, L, D = x.shape
    N = B * L
    assert D == 128 and N % (NC * NS * CH) == 0
    x2 = x.reshape(N, D)
    tagsi = emotion_tags.astype(jnp.int32).reshape(N)
    epack = jnp.stack([emb_table[0], emb_table[1]], axis=-1).reshape(2 * D)
    out = _sc_call(x2, tagsi, epack, ln_gamma, ln_beta)
    return out.reshape(B, L, D)


# speedup vs baseline: 1.6549x; 1.6549x over previous
import functools

import jax
import jax.numpy as jnp
from jax import lax
from jax.experimental import pallas as pl
from jax.experimental.pallas import tpu as pltpu
from jax.experimental.pallas import tpu_sc as plsc

EPS = 1e-12

NC = 2
NS = 16
LN = 16
CH = 128


def _rsqrt_newton(v):
    i = plsc.bitcast(v, jnp.int32)
    y = plsc.bitcast(jnp.int32(0x5F3759DF) - lax.shift_right_arithmetic(i, 1),
                     jnp.float32)
    for _ in range(3):
        y = y * (1.5 - 0.5 * v * y * y)
    return y


def _sc_body(per_w, n_chunks, x_hbm, tags_hbm, epack_hbm, gamma_hbm, beta_hbm,
             out_hbm, xbuf, obuf, tagbuf, ebuf, gbuf, bbuf, scratch):
    wid = lax.axis_index("s") * NC + lax.axis_index("c")
    base = wid * per_w
    pltpu.sync_copy(epack_hbm, ebuf)
    pltpu.sync_copy(gamma_hbm, gbuf)
    pltpu.sync_copy(beta_hbm, bbuf)
    lanes = lax.iota(jnp.int32, LN)

    def chunk_body(ci, _):
        tok0 = base + ci * CH
        pltpu.sync_copy(x_hbm.at[pl.ds(tok0, CH), :], xbuf)
        pltpu.sync_copy(tags_hbm.at[pl.ds(tok0, CH)], tagbuf)

        def group_body(g, _):
            rows = g * LN + lanes
            tagv = plsc.load_gather(tagbuf, [rows])
            zero = jnp.zeros((LN,), jnp.float32)

            def d1_body(d0, carry):
                acc, accsq = carry
                hs = []
                for k in range(8):
                    dfull = jnp.broadcast_to(d0 + k, (LN,))
                    xv = plsc.load_gather(xbuf, [rows, dfull])
                    ev = plsc.load_gather(ebuf, [tagv + (2 * d0 + 2 * k)])
                    hv = xv + ev
                    plsc.store_scatter(scratch, [dfull, lanes], hv)
                    hs.append(hv)
                s = hs
                q = [h * h for h in hs]
                while len(s) > 1:
                    s = [a + b for a, b in zip(s[::2], s[1::2])]
                    q = [a + b for a, b in zip(q[::2], q[1::2])]
                return acc + s[0], accsq + q[0]

            acc, accsq = plsc.parallel_loop(
                0, 128, 8, unroll=2, carry=(zero, zero))(d1_body)
            mean = acc * (1.0 / 128.0)
            var = accsq * (1.0 / 128.0) - mean * mean
            rstd = _rsqrt_newton(var + EPS)

            def d2_body(d0):
                for k in range(8):
                    dfull = jnp.broadcast_to(d0 + k, (LN,))
                    hv = plsc.load_gather(scratch, [dfull, lanes])
                    gd = plsc.load_gather(gbuf, [dfull])
                    bd = plsc.load_gather(bbuf, [dfull])
                    ov = (hv - mean) * rstd * gd + bd
                    plsc.store_scatter(obuf, [rows, dfull], ov)

            plsc.parallel_loop(0, 128, 8, unroll=2)(d2_body)
            return 0

        lax.fori_loop(0, CH // LN, group_body, 0)
        pltpu.sync_copy(obuf, out_hbm.at[pl.ds(tok0, CH), :])
        return 0

    lax.fori_loop(0, n_chunks, chunk_body, 0)


def _sc_call(x2, tagsi, epack, ln_gamma, ln_beta):
    N, D = x2.shape
    NW = NC * NS
    per_w = N // NW
    n_chunks = per_w // CH
    body = functools.partial(_sc_body, per_w, n_chunks)
    f = pl.kernel(
        body,
        mesh=plsc.VectorSubcoreMesh(core_axis_name="c", subcore_axis_name="s"),
        compiler_params=pltpu.CompilerParams(needs_layout_passes=False),
        out_type=jax.ShapeDtypeStruct((N, D), jnp.float32),
        scratch_types=[
            pltpu.VMEM((CH, D), jnp.float32),
            pltpu.VMEM((CH, D), jnp.float32),
            pltpu.VMEM((CH,), jnp.int32),
            pltpu.VMEM((2 * D,), jnp.float32),
            pltpu.VMEM((D,), jnp.float32),
            pltpu.VMEM((D,), jnp.float32),
            pltpu.VMEM((D, LN), jnp.float32),
        ],
    )
    return f(x2, tagsi, epack, ln_gamma, ln_beta)


def kernel(x, emotion_tags, emb_table, ln_gamma, ln_beta):
    B, L, D = x.shape
    N = B * L
    assert D == 128 and N % (NC * NS * CH) == 0
    x2 = x.reshape(N, D)
    tagsi = emotion_tags.astype(jnp.int32).reshape(N)
    epack = jnp.stack([emb_table[0], emb_table[1]], axis=-1).reshape(2 * D)
    out = _sc_call(x2, tagsi, epack, ln_gamma, ln_beta)
    return out.reshape(B, L, D)


# speedup vs baseline: 1.6682x; 1.0080x over previous
import functools

import jax
import jax.numpy as jnp
from jax import lax
from jax.experimental import pallas as pl
from jax.experimental.pallas import tpu as pltpu
from jax.experimental.pallas import tpu_sc as plsc

EPS = 1e-12

NC = 2
NS = 16
LN = 16
CH = 128


def _rsqrt_newton(v):
    i = plsc.bitcast(v, jnp.int32)
    y = plsc.bitcast(jnp.int32(0x5F3759DF) - lax.shift_right_arithmetic(i, 1),
                     jnp.float32)
    for _ in range(3):
        y = y * (1.5 - 0.5 * v * y * y)
    return y


def _sc_body(per_w, n_chunks, x_hbm, tags_hbm, epack_hbm, gamma_hbm, beta_hbm,
             out_hbm, xbuf, obuf, tagbuf, ebuf, gbuf, bbuf, scratch):
    wid = lax.axis_index("s") * NC + lax.axis_index("c")
    base = wid * per_w
    pltpu.sync_copy(epack_hbm, ebuf)
    pltpu.sync_copy(gamma_hbm, gbuf)
    pltpu.sync_copy(beta_hbm, bbuf)
    lanes = lax.iota(jnp.int32, LN)

    def chunk_body(ci, _):
        tok0 = base + ci * CH
        pltpu.sync_copy(x_hbm.at[pl.ds(tok0, CH), :], xbuf)
        pltpu.sync_copy(tags_hbm.at[pl.ds(tok0, CH)], tagbuf)

        def group_body(g, _):
            rows = g * LN + lanes
            tagv = plsc.load_gather(tagbuf, [rows])
            zero = jnp.zeros((LN,), jnp.float32)

            def d1_body(d0, carry):
                acc, accsq = carry
                hs = []
                for k in range(8):
                    dfull = jnp.broadcast_to(d0 + k, (LN,))
                    xv = plsc.load_gather(xbuf, [rows, dfull])
                    ev = plsc.load_gather(ebuf, [tagv + (2 * d0 + 2 * k)])
                    hv = xv + ev
                    plsc.store_scatter(scratch, [dfull, lanes], hv)
                    hs.append(hv)
                s = hs
                q = [h * h for h in hs]
                while len(s) > 1:
                    s = [a + b for a, b in zip(s[::2], s[1::2])]
                    q = [a + b for a, b in zip(q[::2], q[1::2])]
                return acc + s[0], accsq + q[0]

            acc, accsq = plsc.parallel_loop(
                0, 128, 8, unroll=4, carry=(zero, zero))(d1_body)
            mean = acc * (1.0 / 128.0)
            var = accsq * (1.0 / 128.0) - mean * mean
            rstd = _rsqrt_newton(var + EPS)

            def d2_body(d0):
                for k in range(8):
                    dfull = jnp.broadcast_to(d0 + k, (LN,))
                    hv = plsc.load_gather(scratch, [dfull, lanes])
                    gd = plsc.load_gather(gbuf, [dfull])
                    bd = plsc.load_gather(bbuf, [dfull])
                    ov = (hv - mean) * rstd * gd + bd
                    plsc.store_scatter(obuf, [rows, dfull], ov)

            plsc.parallel_loop(0, 128, 8, unroll=4)(d2_body)
            return 0

        lax.fori_loop(0, CH // LN, group_body, 0)
        pltpu.sync_copy(obuf, out_hbm.at[pl.ds(tok0, CH), :])
        return 0

    lax.fori_loop(0, n_chunks, chunk_body, 0)


def _sc_call(x2, tagsi, epack, ln_gamma, ln_beta):
    N, D = x2.shape
    NW = NC * NS
    per_w = N // NW
    n_chunks = per_w // CH
    body = functools.partial(_sc_body, per_w, n_chunks)
    f = pl.kernel(
        body,
        mesh=plsc.VectorSubcoreMesh(core_axis_name="c", subcore_axis_name="s"),
        compiler_params=pltpu.CompilerParams(needs_layout_passes=False),
        out_type=jax.ShapeDtypeStruct((N, D), jnp.float32),
        scratch_types=[
            pltpu.VMEM((CH, D), jnp.float32),
            pltpu.VMEM((CH, D), jnp.float32),
            pltpu.VMEM((CH,), jnp.int32),
            pltpu.VMEM((2 * D,), jnp.float32),
            pltpu.VMEM((D,), jnp.float32),
            pltpu.VMEM((D,), jnp.float32),
            pltpu.VMEM((D, LN), jnp.float32),
        ],
    )
    return f(x2, tagsi, epack, ln_gamma, ln_beta)


def kernel(x, emotion_tags, emb_table, ln_gamma, ln_beta):
    B, L, D = x.shape
    N = B * L
    assert D == 128 and N % (NC * NS * CH) == 0
    x2 = x.reshape(N, D)
    tagsi = emotion_tags.astype(jnp.int32).reshape(N)
    epack = jnp.stack([emb_table[0], emb_table[1]], axis=-1).reshape(2 * D)
    out = _sc_call(x2, tagsi, epack, ln_gamma, ln_beta)
    return out.reshape(B, L, D)


# speedup vs baseline: 3.6590x; 2.1933x over previous
import functools

import jax
import jax.numpy as jnp
from jax import lax
from jax.experimental import pallas as pl
from jax.experimental.pallas import tpu as pltpu
from jax.experimental.pallas import tpu_sc as plsc

EPS = 1e-12

NC = 2
NS = 16
LN = 16
CH = 128
NJ = 8


def _rsqrt_newton(v):
    i = plsc.bitcast(v, jnp.int32)
    y = plsc.bitcast(jnp.int32(0x5F3759DF) - lax.shift_right_arithmetic(i, 1),
                     jnp.float32)
    for _ in range(3):
        y = y * (1.5 - 0.5 * v * y * y)
    return y


def _tree_sum(vals):
    vals = list(vals)
    while len(vals) > 1:
        vals = [a + b for a, b in zip(vals[::2], vals[1::2])]
    return vals[0]


def _sc_body(per_w, n_chunks, x_hbm, tagf_hbm, const_hbm, out_hbm,
             xbuf, obuf, tagbuf, cbuf):
    wid = lax.axis_index("s") * NC + lax.axis_index("c")
    base = wid * per_w
    pltpu.sync_copy(const_hbm, cbuf)
    t0v = [cbuf[pl.ds(j * LN, LN)] for j in range(NJ)]
    dfv = [cbuf[pl.ds(128 + j * LN, LN)] for j in range(NJ)]
    gv = [cbuf[pl.ds(256 + j * LN, LN)] for j in range(NJ)]
    bv = [cbuf[pl.ds(384 + j * LN, LN)] for j in range(NJ)]

    def chunk_body(ci, _):
        tok0 = base + ci * CH
        pltpu.sync_copy(x_hbm.at[pl.ds(tok0, CH), :], xbuf)
        pltpu.sync_copy(tagf_hbm.at[pl.ds(tok0, CH)], tagbuf)

        def tok_body(tb):
            tagv = tagbuf[pl.ds(tb, LN)]
            for i in range(LN):
                t = tb + i
                tf = jnp.broadcast_to(tagv[i], (LN,))
                hs = []
                for j in range(NJ):
                    xj = xbuf[t, pl.ds(j * LN, LN)]
                    hs.append(xj + (t0v[j] + tf * dfv[j]))
                s = _tree_sum(hs)
                q = _tree_sum([h * h for h in hs])
                sumv = jnp.broadcast_to(jnp.sum(s), (LN,))
                sqv = jnp.broadcast_to(jnp.sum(q), (LN,))
                mean = sumv * (1.0 / 128.0)
                var = sqv * (1.0 / 128.0) - mean * mean
                rstd = _rsqrt_newton(var + EPS)
                for j in range(NJ):
                    obuf[t, pl.ds(j * LN, LN)] = (
                        (hs[j] - mean) * rstd * gv[j] + bv[j])

        plsc.parallel_loop(0, CH, LN, unroll=1)(tok_body)
        pltpu.sync_copy(obuf, out_hbm.at[pl.ds(tok0, CH), :])
        return 0

    lax.fori_loop(0, n_chunks, chunk_body, 0)


def _sc_call(x2, tagf, consts):
    N, D = x2.shape
    NW = NC * NS
    per_w = N // NW
    n_chunks = per_w // CH
    body = functools.partial(_sc_body, per_w, n_chunks)
    f = pl.kernel(
        body,
        mesh=plsc.VectorSubcoreMesh(core_axis_name="c", subcore_axis_name="s"),
        compiler_params=pltpu.CompilerParams(needs_layout_passes=False),
        out_type=jax.ShapeDtypeStruct((N, D), jnp.float32),
        scratch_types=[
            pltpu.VMEM((CH, D), jnp.float32),
            pltpu.VMEM((CH, D), jnp.float32),
            pltpu.VMEM((CH,), jnp.float32),
            pltpu.VMEM((4 * D,), jnp.float32),
        ],
    )
    return f(x2, tagf, consts)


def kernel(x, emotion_tags, emb_table, ln_gamma, ln_beta):
    B, L, D = x.shape
    N = B * L
    assert D == 128 and N % (NC * NS * CH) == 0
    x2 = x.reshape(N, D)
    tagf = emotion_tags.astype(jnp.float32).reshape(N)
    consts = jnp.concatenate(
        [emb_table[0], emb_table[1] - emb_table[0], ln_gamma, ln_beta])
    out = _sc_call(x2, tagf, consts)
    return out.reshape(B, L, D)
